# Initial kernel scaffold; baseline (speedup 1.0000x reference)
import jax
import jax.numpy as jnp
from jax.experimental import pallas as pl


def kernel(input_ids, embedding):
    raise NotImplementedError("write your pallas kernel here")


# speedup vs baseline: 7.6781x; 7.6781x over previous
import functools

import jax
import jax.numpy as jnp
from jax import lax
from jax.experimental import pallas as pl
from jax.experimental.pallas import tpu as pltpu
from jax.experimental.pallas import tpu_sc as plsc

VOCAB = 100000
D = 128
B = 1024
S = 200
N = B * S

NC, NS = 2, 16
NW = NC * NS
PER_W = N // NW
IDX_ROWS = PER_W // D
G_PER_CHUNK = 5
CHUNK = G_PER_CHUNK * D
N_CHUNKS = PER_W // CHUNK


def _gather_body(table_hbm, idx_hbm, out_hbm, idx_v, rows_v, sem):
    wid = lax.axis_index("s") * NC + lax.axis_index("c")
    pltpu.sync_copy(idx_hbm.at[wid], idx_v)
    base = wid * PER_W

    def chunk(g, _):
        copies = []
        for j in range(G_PER_CHUNK):
            c = pltpu.async_copy(
                table_hbm.at[idx_v.at[g * G_PER_CHUNK + j]],
                rows_v.at[pl.ds(j * D, D)],
                sem,
            )
            copies.append(c)
        for c in copies:
            c.wait()
        pltpu.sync_copy(rows_v, out_hbm.at[pl.ds(base + g * CHUNK, CHUNK)])
        return _

    lax.fori_loop(0, N_CHUNKS, chunk, None)


@jax.jit
def _gather(ids2d, table):
    run = pl.kernel(
        _gather_body,
        out_type=jax.ShapeDtypeStruct((N, D), jnp.float32),
        mesh=plsc.VectorSubcoreMesh(core_axis_name="c", subcore_axis_name="s"),
        scratch_types=[
            pltpu.VMEM((IDX_ROWS, D), jnp.int32),
            pltpu.VMEM((CHUNK, D), jnp.float32),
            pltpu.SemaphoreType.DMA,
        ],
    )
    return run(table, ids2d)


def kernel(input_ids, embedding):
    ids3d = input_ids.reshape(NW, IDX_ROWS, D).astype(jnp.int32)
    out = _gather(ids3d, embedding)
    return out.reshape(B, S, D)


# speedup vs baseline: 7.7274x; 1.0064x over previous
import jax
import jax.numpy as jnp
from jax import lax
from jax.experimental import pallas as pl
from jax.experimental.pallas import tpu as pltpu
from jax.experimental.pallas import tpu_sc as plsc

VOCAB = 100000
D = 128
B = 1024
S = 200
N = B * S

NC, NS = 2, 16
NW = NC * NS
PER_W = N // NW
IDX_ROWS = PER_W // D
G_PER_CHUNK = 2
CHUNK = G_PER_CHUNK * D
N_CHUNKS = PER_W // CHUNK


def _gather_body(table_hbm, idx_hbm, out_hbm, idx_v, bufs, sem_in, sem_out):
    wid = lax.axis_index("s") * NC + lax.axis_index("c")
    pltpu.sync_copy(idx_hbm.at[wid], idx_v)
    base = wid * PER_W

    def do_chunk(g, s):
        copies = [
            pltpu.async_copy(
                table_hbm.at[idx_v.at[g * G_PER_CHUNK + j]],
                bufs.at[s].at[pl.ds(j * D, D)],
                sem_in,
            )
            for j in range(G_PER_CHUNK)
        ]
        for c in copies:
            c.wait()
        pltpu.async_copy(
            bufs.at[s], out_hbm.at[pl.ds(base + g * CHUNK, CHUNK)], sem_out
        )

    def wait_out(g, s):
        pltpu.make_async_copy(
            bufs.at[s], out_hbm.at[pl.ds(base + g * CHUNK, CHUNK)], sem_out
        ).wait()

    do_chunk(0, 0)
    do_chunk(1, 1)

    def body(g, carry):
        s = g % 2
        wait_out(g - 2, s)
        do_chunk(g, s)
        return carry

    lax.fori_loop(2, N_CHUNKS, body, None)
    wait_out(N_CHUNKS - 2, (N_CHUNKS - 2) % 2)
    wait_out(N_CHUNKS - 1, (N_CHUNKS - 1) % 2)


@jax.jit
def _gather(ids3d, table):
    run = pl.kernel(
        _gather_body,
        out_type=jax.ShapeDtypeStruct((N, D), jnp.float32),
        mesh=plsc.VectorSubcoreMesh(core_axis_name="c", subcore_axis_name="s"),
        scratch_types=[
            pltpu.VMEM((IDX_ROWS, D), jnp.int32),
            pltpu.VMEM((2, CHUNK, D), jnp.float32),
            pltpu.SemaphoreType.DMA,
            pltpu.SemaphoreType.DMA,
        ],
    )
    return run(table, ids3d)


def kernel(input_ids, embedding):
    ids3d = input_ids.reshape(NW, IDX_ROWS, D).astype(jnp.int32)
    out = _gather(ids3d, embedding)
    return out.reshape(B, S, D)


# speedup vs baseline: 8.1278x; 1.0518x over previous
import jax
import jax.numpy as jnp
from jax import lax
from jax.experimental import pallas as pl
from jax.experimental.pallas import tpu as pltpu
from jax.experimental.pallas import tpu_sc as plsc

VOCAB = 100000
D = 128
B = 1024
S = 200
N = B * S

NC, NS = 2, 16
NW = NC * NS
PER_W = N // NW
NG = PER_W // D
NB = 4


def _gather_body(table_hbm, idx_hbm, out_hbm, idx_v, bufs, sem_in, sem_out):
    wid = lax.axis_index("s") * NC + lax.axis_index("c")
    pltpu.sync_copy(idx_hbm.at[wid], idx_v)
    base = wid * PER_W

    def issue_gather(i):
        pltpu.async_copy(table_hbm.at[idx_v.at[i]], bufs.at[i % NB], sem_in)

    def wait_in():
        pltpu.make_async_copy(
            table_hbm.at[pl.ds(0, D)], bufs.at[0], sem_in
        ).wait()

    def start_out(i):
        pltpu.async_copy(
            bufs.at[i % NB], out_hbm.at[pl.ds(base + i * D, D)], sem_out
        )

    def wait_out(i):
        pltpu.make_async_copy(
            bufs.at[i % NB], out_hbm.at[pl.ds(base + i * D, D)], sem_out
        ).wait()

    issue_gather(0)
    issue_gather(1)

    def head(i, carry):
        wait_in()
        start_out(i)
        issue_gather(i + 2)
        return carry

    lax.fori_loop(0, 2, head, None)

    def steady(i, carry):
        wait_in()
        start_out(i)
        wait_out(i - 2)
        issue_gather(i + 2)
        return carry

    lax.fori_loop(2, NG - 2, steady, None)

    def tail(i, carry):
        wait_in()
        start_out(i)
        wait_out(i - 2)
        return carry

    lax.fori_loop(NG - 2, NG, tail, None)
    wait_out(NG - 2)
    wait_out(NG - 1)


@jax.jit
def _gather(ids3d, table):
    run = pl.kernel(
        _gather_body,
        out_type=jax.ShapeDtypeStruct((N, D), jnp.float32),
        mesh=plsc.VectorSubcoreMesh(core_axis_name="c", subcore_axis_name="s"),
        scratch_types=[
            pltpu.VMEM((NG, D), jnp.int32),
            pltpu.VMEM((NB, D, D), jnp.float32),
            pltpu.SemaphoreType.DMA,
            pltpu.SemaphoreType.DMA,
        ],
    )
    return run(table, ids3d)


def kernel(input_ids, embedding):
    ids3d = input_ids.reshape(NW, NG, D).astype(jnp.int32)
    out = _gather(ids3d, embedding)
    return out.reshape(B, S, D)


# speedup vs baseline: 8.1911x; 1.0078x over previous
import jax
import jax.numpy as jnp
from jax import lax
from jax.experimental import pallas as pl
from jax.experimental.pallas import tpu as pltpu
from jax.experimental.pallas import tpu_sc as plsc

VOCAB = 100000
D = 128
B = 1024
S = 200
N = B * S

NC, NS = 2, 16
NW = NC * NS
PER_W = N // NW
NG = PER_W // D
NB = 6
LA = 4


def _gather_body(table_hbm, idx_hbm, out_hbm, idx_v, bufs, sem_in, sem_out):
    wid = lax.axis_index("s") * NC + lax.axis_index("c")
    pltpu.sync_copy(idx_hbm.at[wid], idx_v)
    base = wid * PER_W

    def issue_gather(i):
        pltpu.async_copy(table_hbm.at[idx_v.at[i]], bufs.at[i % NB], sem_in)

    def wait_in():
        pltpu.make_async_copy(
            table_hbm.at[pl.ds(0, D)], bufs.at[0], sem_in
        ).wait()

    def start_out(i):
        pltpu.async_copy(
            bufs.at[i % NB], out_hbm.at[pl.ds(base + i * D, D)], sem_out
        )

    def wait_out(i):
        pltpu.make_async_copy(
            bufs.at[i % NB], out_hbm.at[pl.ds(base + i * D, D)], sem_out
        ).wait()

    for i in range(LA):
        issue_gather(i)

    def head(i, carry):
        wait_in()
        start_out(i)
        issue_gather(i + LA)
        return carry

    lax.fori_loop(0, NB - LA, head, None)

    def steady(i, carry):
        wait_in()
        start_out(i)
        wait_out(i + LA - NB)
        issue_gather(i + LA)
        return carry

    lax.fori_loop(NB - LA, NG - LA, steady, None)

    def tail(i, carry):
        wait_in()
        start_out(i)
        wait_out(i + LA - NB)
        return carry

    lax.fori_loop(NG - LA, NG, tail, None)
    for i in range(NG + LA - NB, NG):
        wait_out(i)


@jax.jit
def _gather(ids3d, table):
    run = pl.kernel(
        _gather_body,
        out_type=jax.ShapeDtypeStruct((N, D), jnp.float32),
        mesh=plsc.VectorSubcoreMesh(core_axis_name="c", subcore_axis_name="s"),
        scratch_types=[
            pltpu.VMEM((NG, D), jnp.int32),
            pltpu.VMEM((NB, D, D), jnp.float32),
            pltpu.SemaphoreType.DMA,
            pltpu.SemaphoreType.DMA,
        ],
    )
    return run(table, ids3d)


def kernel(input_ids, embedding):
    ids3d = input_ids.reshape(NW, NG, D).astype(jnp.int32)
    out = _gather(ids3d, embedding)
    return out.reshape(B, S, D)
